# Initial kernel scaffold; baseline (speedup 1.0000x reference)
import jax
import jax.numpy as jnp
from jax.experimental import pallas as pl


def kernel(x, emb, W_lin, b_lin):
    raise NotImplementedError("write your pallas kernel here")


# speedup vs baseline: 77.1721x; 77.1721x over previous
import functools

import jax
import jax.numpy as jnp
from jax import lax
from jax.experimental import pallas as pl
from jax.experimental.pallas import tpu as pltpu
from jax.experimental.pallas import tpu_sc as plsc

B = 1024
J = 2600
JP = 2608
V = 2600
VP = 2601
F = 16
L = 16
NW = 32
ROWS_PER_TILE = B // NW
CHUNK_ROWS = 8
NCHUNK = ROWS_PER_TILE // CHUNK_ROWS
NGROUP = JP // L


def _sc_stage1(xp, embT, wp, xbf):
    mesh = plsc.VectorSubcoreMesh(core_axis_name="c", subcore_axis_name="s")

    @functools.partial(
        pl.kernel,
        out_type=(
            jax.ShapeDtypeStruct((B, F * L), jnp.float32),
            jax.ShapeDtypeStruct((B, F * L), jnp.float32),
            jax.ShapeDtypeStruct((B, L), jnp.float32),
        ),
        mesh=mesh,
        compiler_params=pltpu.CompilerParams(needs_layout_passes=False),
        scratch_types=[
            pltpu.VMEM((F * VP,), jnp.float32),
            pltpu.VMEM((JP,), jnp.float32),
            pltpu.VMEM((VP + 7,), jnp.float32),
            pltpu.VMEM((2, CHUNK_ROWS, JP), jnp.int32),
            pltpu.VMEM((ROWS_PER_TILE, F * L), jnp.float32),
            pltpu.VMEM((ROWS_PER_TILE, F * L), jnp.float32),
            pltpu.VMEM((ROWS_PER_TILE, L), jnp.float32),
            pltpu.SemaphoreType.DMA,
            pltpu.SemaphoreType.DMA,
            pltpu.SemaphoreType.DMA,
        ],
    )
    def k(x_hbm, embT_hbm, w_hbm, xbf_hbm, s_hbm, q_hbm, lin_hbm,
          embT_v, w_v, xbf_v, xbuf, sbuf, qbuf, linbuf, sem_t, sem_a, sem_b):
        wid = lax.axis_index("s") * 2 + lax.axis_index("c")
        base = wid * ROWS_PER_TILE

        cp_t = pltpu.async_copy(embT_hbm, embT_v, sem_t)
        cp_w = pltpu.async_copy(w_hbm, w_v, sem_t)
        cp_x = pltpu.async_copy(xbf_hbm, xbf_v, sem_t)
        sems = (sem_a, sem_b)
        cps = [None, None]
        cps[0] = pltpu.async_copy(
            x_hbm.at[pl.ds(base, CHUNK_ROWS), :], xbuf.at[0], sems[0])
        cp_t.wait()
        cp_w.wait()
        cp_x.wait()

        def do_row(rr, chunk_buf, r_in_chunk):
            zero = jnp.zeros((L,), jnp.float32)
            init = tuple(zero for _ in range(2 * F + 1))

            def g_body(g, carry):
                off = pl.multiple_of(g * L, L)
                idx = chunk_buf[r_in_chunk, pl.ds(off, L)]
                w = w_v[pl.ds(off, L)]
                xf = plsc.load_gather(xbf_v, [idx])
                lin = carry[2 * F] + xf * w
                addr = idx
                acc = list(carry)
                for f in range(F):
                    vals = plsc.load_gather(embT_v, [addr])
                    acc[f] = acc[f] + vals
                    acc[F + f] = acc[F + f] + vals * vals
                    if f + 1 < F:
                        addr = addr + VP
                acc[2 * F] = lin
                return tuple(acc)

            fin = lax.fori_loop(0, NGROUP, g_body, init)
            for f in range(F):
                sbuf[rr, pl.ds(f * L, L)] = fin[f]
                qbuf[rr, pl.ds(f * L, L)] = fin[F + f]
            linbuf[rr, :] = fin[2 * F]

        for c in range(NCHUNK):
            if c + 1 < NCHUNK:
                cps[(c + 1) % 2] = pltpu.async_copy(
                    x_hbm.at[pl.ds(base + (c + 1) * CHUNK_ROWS, CHUNK_ROWS), :],
                    xbuf.at[(c + 1) % 2], sems[(c + 1) % 2])
            cps[c % 2].wait()

            def row_body(r, _, c=c):
                do_row(c * CHUNK_ROWS + r, xbuf.at[c % 2], r)
                return 0

            lax.fori_loop(0, CHUNK_ROWS, row_body, 0)

        pltpu.sync_copy(sbuf, s_hbm.at[pl.ds(base, ROWS_PER_TILE), :])
        pltpu.sync_copy(qbuf, q_hbm.at[pl.ds(base, ROWS_PER_TILE), :])
        pltpu.sync_copy(linbuf, lin_hbm.at[pl.ds(base, ROWS_PER_TILE), :])

    return k(xp, embT, wp, xbf)


def _tc_stage2_body(s_ref, q_ref, lin_ref, b_ref, o_ref):
    sel_r = lax.broadcasted_iota(jnp.int32, (F * L, F), 0) // L
    sel_c = lax.broadcasted_iota(jnp.int32, (F * L, F), 1)
    sel = (sel_r == sel_c).astype(jnp.float32)
    s = jnp.dot(s_ref[...], sel, preferred_element_type=jnp.float32,
                precision=lax.Precision.HIGHEST)
    q = jnp.dot(q_ref[...], sel, preferred_element_type=jnp.float32,
                precision=lax.Precision.HIGHEST)
    s2 = s * s
    m1 = jnp.max(s2)
    m2 = jnp.max(q)
    inter = 0.5 * (jnp.sum(s2, axis=1, keepdims=True) / m1
                   - jnp.sum(q, axis=1, keepdims=True) / m2)
    lin = jnp.sum(lin_ref[...], axis=1, keepdims=True) + b_ref[0, 0]
    o_ref[...] = jax.nn.sigmoid(lin + inter)


def _tc_stage2(s_part, q_part, lin_part, b_lin):
    return pl.pallas_call(
        _tc_stage2_body,
        out_shape=jax.ShapeDtypeStruct((B, 1), jnp.float32),
    )(s_part, q_part, lin_part, b_lin.reshape(1, 1))


def kernel(x, emb, W_lin, b_lin):
    xp = jnp.pad(x.astype(jnp.int32), ((0, 0), (0, JP - J)),
                 constant_values=V)
    emb_pad = jnp.concatenate(
        [emb, jnp.zeros((1, F), jnp.float32)], axis=0)
    embT = emb_pad.T.reshape(-1)
    w_bf = W_lin.reshape(-1).astype(jnp.bfloat16).astype(jnp.float32)
    wp = jnp.concatenate(
        [w_bf, jnp.zeros((JP - J,), jnp.float32)])
    xbf = jnp.pad(
        jnp.arange(VP, dtype=jnp.float32).astype(jnp.bfloat16).astype(
            jnp.float32), (0, 7))
    s_part, q_part, lin_part = _sc_stage1(xp, embT, wp, xbf)
    out = _tc_stage2(s_part, q_part, lin_part, b_lin)
    return jnp.squeeze(out, axis=1)
